# Initial kernel scaffold; baseline (speedup 1.0000x reference)
import functools

import jax
import jax.numpy as jnp
from jax import lax
from jax.experimental import pallas as pl
from jax.experimental.pallas import tpu as pltpu
from jax.experimental.pallas import tpu_sc as plsc

N = 50000
E = 800000
G = 1024
D = 64
H = 128
L = 6

NSC = 2
NSUB = 16
NW = NSC * NSUB

NP = 50176
TPT = NP // NW
HALF = NP // 2
TRASH = HALF
ACC_ROWS = HALF + 16
EPT = E // NSUB
SET_CH = 2000
PADM = 1024
W_C = NP
W_CB = W_C + PADM
NBLK = W_C // 128
CH = 128
EMB_CH = 112
PG_ROWS = G + 16


def _mesh():
    return plsc.VectorSubcoreMesh(
        core_axis_name="c", subcore_axis_name="s",
        num_cores=NSC, num_subcores=NSUB)


def _setup_body(src_hbm, dst_hbm, srcc_hbm, dstc_hbm, cnt_hbm,
                sbuf, dbuf, so, do, cbuf):
    c = lax.axis_index("c")
    s = lax.axis_index("s")
    r = c * NSUB + s
    lo = c * HALF
    ebase = s * EPT

    def chunk_body(k, ptr):
        pltpu.sync_copy(src_hbm.at[pl.ds(ebase + k * SET_CH, SET_CH)], sbuf)
        pltpu.sync_copy(dst_hbm.at[pl.ds(ebase + k * SET_CH, SET_CH)], dbuf)

        def vec_body(i, ptr):
            sv = sbuf[pl.ds(i * 16, 16)]
            dv = dbuf[pl.ds(i * 16, 16)]
            ldv = dv - lo
            mask = (ldv >= 0) & (ldv < HALF)
            plsc.store_compressed(so.at[pl.ds(ptr, 16)], sv, mask=mask)
            plsc.store_compressed(do.at[pl.ds(ptr, 16)], ldv, mask=mask)
            return ptr + jnp.sum(mask.astype(jnp.int32))

        return lax.fori_loop(0, SET_CH // 16, vec_body, ptr)

    ptr = lax.fori_loop(0, EPT // SET_CH, chunk_body, jnp.int32(0))

    zeros16 = jnp.zeros((16,), jnp.int32)
    trash16 = jnp.full((16,), TRASH, jnp.int32)

    def pad_body(k, carry):
        so[pl.ds(ptr + k * 16, 16)] = zeros16
        do[pl.ds(ptr + k * 16, 16)] = trash16
        return carry

    lax.fori_loop(0, PADM // 16, pad_body, 0)
    padded = ((ptr + PADM - 1) // PADM) * PADM
    cbuf[...] = jnp.full((16,), padded, jnp.int32)

    pltpu.sync_copy(so.at[pl.ds(0, W_C)], srcc_hbm.at[r])
    pltpu.sync_copy(do.at[pl.ds(0, W_C)], dstc_hbm.at[r])
    pltpu.sync_copy(cbuf, cnt_hbm.at[r])


def _setup(src, dst):
    k = pl.kernel(
        _setup_body,
        out_type=(jax.ShapeDtypeStruct((NW, W_C), jnp.int32),
                  jax.ShapeDtypeStruct((NW, W_C), jnp.int32),
                  jax.ShapeDtypeStruct((NW, 16), jnp.int32)),
        mesh=_mesh(),
        scratch_types=[pltpu.VMEM((SET_CH,), jnp.int32),
                       pltpu.VMEM((SET_CH,), jnp.int32),
                       pltpu.VMEM((W_CB,), jnp.int32),
                       pltpu.VMEM((W_CB,), jnp.int32),
                       pltpu.VMEM((16,), jnp.int32)],
    )
    return k(src, dst)


def _embed_body(x_hbm, wemb_hbm, h_hbm, xi, rows, sem):
    c = lax.axis_index("c")
    s = lax.axis_index("s")
    base = (c * NSUB + s) * TPT

    @pl.loop(0, TPT // EMB_CH)
    def _(g):
        off = base + g * EMB_CH
        pltpu.sync_copy(x_hbm.at[pl.ds(off, EMB_CH)], xi)
        pltpu.async_copy(wemb_hbm.at[xi], rows, sem).wait()
        pltpu.sync_copy(rows, h_hbm.at[pl.ds(off, EMB_CH)])


def _embed(x_pad, wemb):
    k = pl.kernel(
        _embed_body,
        out_type=jax.ShapeDtypeStruct((NP, D), jnp.float32),
        mesh=_mesh(),
        scratch_types=[pltpu.VMEM((EMB_CH,), jnp.int32),
                       pltpu.VMEM((EMB_CH, D), jnp.float32),
                       pltpu.SemaphoreType.DMA],
    )
    return k(x_pad, wemb)


def _aggr_body(h_hbm, srcc_hbm, dstc_hbm, cnt_hbm, zeros_hbm, out_hbm,
               acc, sidx, didx, rows, cnt, gsem0, gsem1, asem):
    c = lax.axis_index("c")
    s = lax.axis_index("s")
    r = c * NSUB + s

    pltpu.sync_copy(zeros_hbm, acc.at[pl.ds(s * TPT, TPT)])
    plsc.subcore_barrier()

    pltpu.sync_copy(cnt_hbm.at[r], cnt)
    npairs = jnp.max(cnt[...]) // PADM

    def pair_body(p, carry):
        blk = p * 8
        pltpu.sync_copy(srcc_hbm.at[r, pl.ds(blk, 4)], sidx.at[0])
        pltpu.sync_copy(dstc_hbm.at[r, pl.ds(blk, 4)], didx.at[0])
        for j in range(4):
            pltpu.async_copy(h_hbm.at[sidx.at[0, j]], rows.at[0, j], gsem0)
        pltpu.sync_copy(srcc_hbm.at[r, pl.ds(blk + 4, 4)], sidx.at[1])
        pltpu.sync_copy(dstc_hbm.at[r, pl.ds(blk + 4, 4)], didx.at[1])
        for j in range(4):
            pltpu.async_copy(h_hbm.at[sidx.at[1, j]], rows.at[1, j], gsem1)
        for j in range(4):
            pltpu.make_async_copy(h_hbm.at[sidx.at[0, j]], rows.at[0, j],
                                  gsem0).wait()
        for j in range(4):
            pltpu.async_copy(rows.at[0, j], acc.at[didx.at[0, j]], asem,
                             add=True)
        for j in range(4):
            pltpu.make_async_copy(h_hbm.at[sidx.at[1, j]], rows.at[1, j],
                                  gsem1).wait()
        for j in range(4):
            pltpu.async_copy(rows.at[1, j], acc.at[didx.at[1, j]], asem,
                             add=True)
        for half in range(2):
            for j in range(4):
                pltpu.make_async_copy(rows.at[half, j],
                                      acc.at[didx.at[half, j]], asem).wait()
        return carry

    lax.fori_loop(0, npairs, pair_body, 0)
    plsc.subcore_barrier()
    pltpu.sync_copy(acc.at[pl.ds(s * TPT, TPT)],
                    out_hbm.at[pl.ds(c * HALF + s * TPT, TPT)])


def _aggregate(h, srcc, dstc, counts, zeros_rows):
    k = pl.kernel(
        _aggr_body,
        out_type=jax.ShapeDtypeStruct((NP, D), jnp.float32),
        mesh=_mesh(),
        scratch_types=[pltpu.VMEM_SHARED((ACC_ROWS, D), jnp.float32),
                       pltpu.VMEM((2, 4, CH), jnp.int32),
                       pltpu.VMEM((2, 4, CH), jnp.int32),
                       pltpu.VMEM((2, 4, CH, D), jnp.float32),
                       pltpu.VMEM((16,), jnp.int32),
                       pltpu.SemaphoreType.DMA,
                       pltpu.SemaphoreType.DMA,
                       pltpu.SemaphoreType.DMA],
    )
    srcc3 = srcc.reshape(NW, NBLK, CH)
    dstc3 = dstc.reshape(NW, NBLK, CH)
    return k(h, srcc3, dstc3, counts, zeros_rows)


def _pool_body(h_hbm, batch_hbm, zeros_hbm, pp_hbm, pooled, bidx, rows, sem):
    c = lax.axis_index("c")
    s = lax.axis_index("s")
    base = c * HALF + s * TPT

    pltpu.sync_copy(zeros_hbm.at[pl.ds(0, G // NSUB)],
                    pooled.at[pl.ds(s * (G // NSUB), G // NSUB)])
    plsc.subcore_barrier()

    @pl.loop(0, TPT // EMB_CH)
    def _(g):
        off = base + g * EMB_CH
        pltpu.sync_copy(batch_hbm.at[pl.ds(off, EMB_CH)], bidx)
        pltpu.sync_copy(h_hbm.at[pl.ds(off, EMB_CH)], rows)
        pltpu.sync_copy(rows, pooled.at[bidx], add=True)

    plsc.subcore_barrier()
    pltpu.sync_copy(pooled.at[pl.ds(s * (G // NSUB), G // NSUB)],
                    pp_hbm.at[c, pl.ds(s * (G // NSUB), G // NSUB)])


def _pool(h, batch_pad, zeros_rows):
    k = pl.kernel(
        _pool_body,
        out_type=jax.ShapeDtypeStruct((NSC, G, D), jnp.float32),
        mesh=_mesh(),
        scratch_types=[pltpu.VMEM_SHARED((PG_ROWS, D), jnp.float32),
                       pltpu.VMEM((EMB_CH,), jnp.int32),
                       pltpu.VMEM((EMB_CH, D), jnp.float32),
                       pltpu.SemaphoreType.DMA],
    )
    return k(h, batch_pad, zeros_rows)


def _conv_body(a_ref, h_ref, wr_ref, wo_ref, b_ref, o_ref):
    o_ref[...] = jnp.maximum(
        jnp.dot(a_ref[...], wr_ref[...],
                preferred_element_type=jnp.float32,
                precision=lax.Precision.HIGHEST)
        + jnp.dot(h_ref[...], wo_ref[...],
                  preferred_element_type=jnp.float32,
                  precision=lax.Precision.HIGHEST)
        + b_ref[...], 0.0)


def _conv_tc(aggr, h, wr, wo, b):
    blk = 6272
    return pl.pallas_call(
        _conv_body,
        grid=(NP // blk,),
        in_specs=[pl.BlockSpec((blk, D), lambda i: (i, 0)),
                  pl.BlockSpec((blk, D), lambda i: (i, 0)),
                  pl.BlockSpec((D, D), lambda i: (0, 0)),
                  pl.BlockSpec((D, D), lambda i: (0, 0)),
                  pl.BlockSpec((1, D), lambda i: (0, 0))],
        out_specs=pl.BlockSpec((blk, D), lambda i: (i, 0)),
        out_shape=jax.ShapeDtypeStruct((NP, D), jnp.float32),
    )(aggr, h, wr, wo, b.reshape(1, D))


def _mlp_body(pp_ref, w0_ref, b0_ref, wl_ref, bl_ref, wout_ref, bout_ref,
              o_ref):
    p = pp_ref[0] + pp_ref[1]
    hh = jnp.maximum(
        jnp.dot(p, w0_ref[...], preferred_element_type=jnp.float32,
                precision=lax.Precision.HIGHEST) + b0_ref[...], 0.0)
    for i in range(2):
        hh = jnp.maximum(
            jnp.dot(hh, wl_ref[i], preferred_element_type=jnp.float32,
                    precision=lax.Precision.HIGHEST) + bl_ref[i], 0.0)
    o_ref[...] = (jnp.dot(hh, wout_ref[...],
                          preferred_element_type=jnp.float32,
                          precision=lax.Precision.HIGHEST)
                  + bout_ref[...])


def _mlp_tc(pp, w0, b0, wl, bl, wout, bout):
    return pl.pallas_call(
        _mlp_body,
        out_shape=jax.ShapeDtypeStruct((G, 1), jnp.float32),
    )(pp, w0, b0.reshape(1, H), wl, bl.reshape(2, 1, H), wout,
      bout.reshape(1, 1))


def kernel(x, edge_index, batch, Wemb, Wrel, Wroot, bconv, W0, b0, Wl, bl,
           Wout, bout):
    x_pad = jnp.concatenate(
        [x.astype(jnp.int32), jnp.zeros((NP - N,), jnp.int32)])
    batch_pad = jnp.concatenate(
        [batch.astype(jnp.int32), jnp.full((NP - N,), G, jnp.int32)])
    src = edge_index[0].astype(jnp.int32)
    dst = edge_index[1].astype(jnp.int32)

    srcc, dstc, counts = _setup(src, dst)
    zeros_rows = jnp.zeros((TPT, D), jnp.float32)

    h = _embed(x_pad, Wemb)
    for i in range(L):
        aggr = _aggregate(h, srcc, dstc, counts, zeros_rows)
        h = _conv_tc(aggr, h, Wrel[i], Wroot[i], bconv[i])

    pp = _pool(h, batch_pad, zeros_rows)
    return _mlp_tc(pp, W0, b0, Wl, bl, Wout, bout)


# speedup vs baseline: 2.9505x; 2.9505x over previous
import dataclasses

import jax
import jax.numpy as jnp
from jax import lax
from jax.experimental import pallas as pl
from jax.experimental.pallas import tpu as pltpu
from jax.experimental.pallas import tpu_sc as plsc

N = 50000
E = 800000
G = 1024
D = 64
H = 128
L = 6

NSC = 2
NSUB = 16
NW = NSC * NSUB

NP = 50176
TPT = NP // NW
QSIZE = NP // 4
QPT = QSIZE // NSUB
TRASH = QSIZE
ACC_ROWS = QSIZE + 16
EPT = E // NSUB
SET_CH = 2000
PADM = 1024
FBUF = PADM + 32
W_Q = 50176
NBLKQ = W_Q // 128
CH = 128
EMB_CH = 112
PG_ROWS = G + 16


def _mesh():
    return plsc.VectorSubcoreMesh(
        core_axis_name="c", subcore_axis_name="s",
        num_cores=NSC, num_subcores=NSUB)


def _sc_params():
    return dataclasses.replace(
        pltpu.CompilerParams(), needs_layout_passes=False,
        use_tc_tiling_on_sc=False)


def _setup_body(src_hbm, dst_hbm, srcc_hbm, dstc_hbm, cnt_hbm,
                sbuf, dbuf, so0, do0, so1, do1, cbuf):
    c = lax.axis_index("c")
    s = lax.axis_index("s")
    r = c * NSUB + s
    base0 = (2 * c) * QSIZE
    base1 = (2 * c + 1) * QSIZE
    ebase = s * EPT

    def chunk_body(k, carry):
        pltpu.sync_copy(src_hbm.at[pl.ds(ebase + k * SET_CH, SET_CH)], sbuf)
        pltpu.sync_copy(dst_hbm.at[pl.ds(ebase + k * SET_CH, SET_CH)], dbuf)

        def vec_body(i, carry):
            ptr0, ptr1, cur0, cur1 = carry
            sv = sbuf[pl.ds(i * 16, 16)]
            dv = dbuf[pl.ds(i * 16, 16)]

            ld0 = dv - base0
            m0 = (ld0 >= 0) & (ld0 < QSIZE)
            plsc.store_compressed(so0.at[pl.ds(ptr0, 16)], sv, mask=m0)
            plsc.store_compressed(do0.at[pl.ds(ptr0, 16)], ld0, mask=m0)
            ptr0 = ptr0 + jnp.sum(m0.astype(jnp.int32))

            ld1 = dv - base1
            m1 = (ld1 >= 0) & (ld1 < QSIZE)
            plsc.store_compressed(so1.at[pl.ds(ptr1, 16)], sv, mask=m1)
            plsc.store_compressed(do1.at[pl.ds(ptr1, 16)], ld1, mask=m1)
            ptr1 = ptr1 + jnp.sum(m1.astype(jnp.int32))

            f0 = ptr0 >= PADM

            @pl.when(f0)
            def _():
                c0 = pl.multiple_of(cur0, PADM)
                pltpu.sync_copy(so0.at[pl.ds(0, PADM)],
                                srcc_hbm.at[r, 0, pl.ds(c0, PADM)])
                pltpu.sync_copy(do0.at[pl.ds(0, PADM)],
                                dstc_hbm.at[r, 0, pl.ds(c0, PADM)])
                so0[pl.ds(0, 16)] = so0[pl.ds(PADM, 16)]
                do0[pl.ds(0, 16)] = do0[pl.ds(PADM, 16)]

            ptr0 = jnp.where(f0, ptr0 - PADM, ptr0)
            cur0 = jnp.where(f0, cur0 + PADM, cur0)

            f1 = ptr1 >= PADM

            @pl.when(f1)
            def _():
                c1 = pl.multiple_of(cur1, PADM)
                pltpu.sync_copy(so1.at[pl.ds(0, PADM)],
                                srcc_hbm.at[r, 1, pl.ds(c1, PADM)])
                pltpu.sync_copy(do1.at[pl.ds(0, PADM)],
                                dstc_hbm.at[r, 1, pl.ds(c1, PADM)])
                so1[pl.ds(0, 16)] = so1[pl.ds(PADM, 16)]
                do1[pl.ds(0, 16)] = do1[pl.ds(PADM, 16)]

            ptr1 = jnp.where(f1, ptr1 - PADM, ptr1)
            cur1 = jnp.where(f1, cur1 + PADM, cur1)

            return ptr0, ptr1, cur0, cur1

        return lax.fori_loop(0, SET_CH // 16, vec_body, carry)

    z = jnp.int32(0)
    ptr0, ptr1, cur0, cur1 = lax.fori_loop(
        0, EPT // SET_CH, chunk_body, (z, z, z, z))

    lanes = lax.iota(jnp.int32, 16)
    zeros16 = jnp.zeros((16,), jnp.int32)
    trash16 = jnp.full((16,), TRASH, jnp.int32)

    def tail_body(k, _):
        pos = lanes + k * 16
        sc0 = so0[pl.ds(k * 16, 16)]
        dc0 = do0[pl.ds(k * 16, 16)]
        so0[pl.ds(k * 16, 16)] = jnp.where(pos >= ptr0, zeros16, sc0)
        do0[pl.ds(k * 16, 16)] = jnp.where(pos >= ptr0, trash16, dc0)
        sc1 = so1[pl.ds(k * 16, 16)]
        dc1 = do1[pl.ds(k * 16, 16)]
        so1[pl.ds(k * 16, 16)] = jnp.where(pos >= ptr1, zeros16, sc1)
        do1[pl.ds(k * 16, 16)] = jnp.where(pos >= ptr1, trash16, dc1)
        return 0

    lax.fori_loop(0, PADM // 16, tail_body, 0)

    cur0 = pl.multiple_of(cur0, PADM)
    cur1 = pl.multiple_of(cur1, PADM)
    pltpu.sync_copy(so0.at[pl.ds(0, PADM)],
                    srcc_hbm.at[r, 0, pl.ds(cur0, PADM)])
    pltpu.sync_copy(do0.at[pl.ds(0, PADM)],
                    dstc_hbm.at[r, 0, pl.ds(cur0, PADM)])
    pltpu.sync_copy(so1.at[pl.ds(0, PADM)],
                    srcc_hbm.at[r, 1, pl.ds(cur1, PADM)])
    pltpu.sync_copy(do1.at[pl.ds(0, PADM)],
                    dstc_hbm.at[r, 1, pl.ds(cur1, PADM)])

    cbuf[...] = jnp.full((16,), cur0 + PADM, jnp.int32)
    pltpu.sync_copy(cbuf, cnt_hbm.at[r, 0])
    cbuf[...] = jnp.full((16,), cur1 + PADM, jnp.int32)
    pltpu.sync_copy(cbuf, cnt_hbm.at[r, 1])


def _setup(src, dst):
    k = pl.kernel(
        _setup_body,
        out_type=(jax.ShapeDtypeStruct((NW, 2, W_Q), jnp.int32),
                  jax.ShapeDtypeStruct((NW, 2, W_Q), jnp.int32),
                  jax.ShapeDtypeStruct((NW, 2, 16), jnp.int32)),
        mesh=_mesh(),
        scratch_types=[pltpu.VMEM((SET_CH,), jnp.int32),
                       pltpu.VMEM((SET_CH,), jnp.int32),
                       pltpu.VMEM((FBUF,), jnp.int32),
                       pltpu.VMEM((FBUF,), jnp.int32),
                       pltpu.VMEM((FBUF,), jnp.int32),
                       pltpu.VMEM((FBUF,), jnp.int32),
                       pltpu.VMEM((16,), jnp.int32)],
        compiler_params=_sc_params(),
    )
    return k(src, dst)


def _embed_body(x_hbm, wemb_hbm, h_hbm, xi, rows, sem):
    c = lax.axis_index("c")
    s = lax.axis_index("s")
    base = (c * NSUB + s) * TPT

    @pl.loop(0, TPT // EMB_CH)
    def _(g):
        off = base + g * EMB_CH
        pltpu.sync_copy(x_hbm.at[pl.ds(off, EMB_CH)], xi)
        pltpu.async_copy(wemb_hbm.at[xi], rows, sem).wait()
        pltpu.sync_copy(rows, h_hbm.at[pl.ds(off, EMB_CH)])


def _embed(x_pad, wemb):
    k = pl.kernel(
        _embed_body,
        out_type=jax.ShapeDtypeStruct((NP, D), jnp.float32),
        mesh=_mesh(),
        scratch_types=[pltpu.VMEM((EMB_CH,), jnp.int32),
                       pltpu.VMEM((EMB_CH, D), jnp.float32),
                       pltpu.SemaphoreType.DMA],
        compiler_params=_sc_params(),
    )
    return k(x_pad, wemb)


def _aggr_body(h_hbm, srcc_hbm, dstc_hbm, cnt_hbm, zeros_hbm, out_hbm,
               acc, sidx, didx, rows, cnt, gsem0, gsem1, asem):
    c = lax.axis_index("c")
    s = lax.axis_index("s")
    r = c * NSUB + s

    for q in range(2):
        pltpu.sync_copy(zeros_hbm.at[pl.ds(0, QPT)],
                        acc.at[pl.ds(s * QPT, QPT)])
        plsc.subcore_barrier()

        pltpu.sync_copy(cnt_hbm.at[r, q], cnt)
        npairs = jnp.max(cnt[...]) // PADM

        def pair_body(p, carry):
            blk = p * 8
            pltpu.sync_copy(srcc_hbm.at[r, q, pl.ds(blk, 4)], sidx.at[0])
            pltpu.sync_copy(dstc_hbm.at[r, q, pl.ds(blk, 4)], didx.at[0])
            for j in range(4):
                pltpu.async_copy(h_hbm.at[sidx.at[0, j]], rows.at[0, j],
                                 gsem0)
            pltpu.sync_copy(srcc_hbm.at[r, q, pl.ds(blk + 4, 4)], sidx.at[1])
            pltpu.sync_copy(dstc_hbm.at[r, q, pl.ds(blk + 4, 4)], didx.at[1])
            for j in range(4):
                pltpu.async_copy(h_hbm.at[sidx.at[1, j]], rows.at[1, j],
                                 gsem1)
            for j in range(4):
                pltpu.make_async_copy(h_hbm.at[sidx.at[0, j]],
                                      rows.at[0, j], gsem0).wait()
            for j in range(4):
                pltpu.async_copy(rows.at[0, j], acc.at[didx.at[0, j]],
                                 asem, add=True)
            for j in range(4):
                pltpu.make_async_copy(h_hbm.at[sidx.at[1, j]],
                                      rows.at[1, j], gsem1).wait()
            for j in range(4):
                pltpu.async_copy(rows.at[1, j], acc.at[didx.at[1, j]],
                                 asem, add=True)
            for half in range(2):
                for j in range(4):
                    pltpu.make_async_copy(rows.at[half, j],
                                          acc.at[didx.at[half, j]],
                                          asem).wait()
            return carry

        lax.fori_loop(0, npairs, pair_body, 0)
        plsc.subcore_barrier()
        pltpu.sync_copy(
            acc.at[pl.ds(s * QPT, QPT)],
            out_hbm.at[pl.ds((2 * c + q) * QSIZE + s * QPT, QPT)])


def _aggregate(h, srcc, dstc, counts, zeros_rows):
    k = pl.kernel(
        _aggr_body,
        out_type=jax.ShapeDtypeStruct((NP, D), jnp.float32),
        mesh=_mesh(),
        scratch_types=[pltpu.VMEM_SHARED((ACC_ROWS, D), jnp.float32),
                       pltpu.VMEM((2, 4, CH), jnp.int32),
                       pltpu.VMEM((2, 4, CH), jnp.int32),
                       pltpu.VMEM((2, 4, CH, D), jnp.float32),
                       pltpu.VMEM((16,), jnp.int32),
                       pltpu.SemaphoreType.DMA,
                       pltpu.SemaphoreType.DMA,
                       pltpu.SemaphoreType.DMA],
        compiler_params=_sc_params(),
    )
    srcc4 = srcc.reshape(NW, 2, NBLKQ, CH)
    dstc4 = dstc.reshape(NW, 2, NBLKQ, CH)
    return k(h, srcc4, dstc4, counts, zeros_rows)


def _pool_body(h_hbm, batch_hbm, zeros_hbm, pp_hbm, pooled, bidx, rows, sem):
    c = lax.axis_index("c")
    s = lax.axis_index("s")
    base = c * (NP // 2) + s * TPT
    gpt = G // NSUB

    pltpu.sync_copy(zeros_hbm.at[pl.ds(0, gpt)],
                    pooled.at[pl.ds(s * gpt, gpt)])
    plsc.subcore_barrier()

    @pl.loop(0, TPT // EMB_CH)
    def _(g):
        off = base + g * EMB_CH
        pltpu.sync_copy(batch_hbm.at[pl.ds(off, EMB_CH)], bidx)
        pltpu.sync_copy(h_hbm.at[pl.ds(off, EMB_CH)], rows)
        pltpu.sync_copy(rows, pooled.at[bidx], add=True)

    plsc.subcore_barrier()
    pltpu.sync_copy(pooled.at[pl.ds(s * gpt, gpt)],
                    pp_hbm.at[c, pl.ds(s * gpt, gpt)])


def _pool(h, batch_pad, zeros_rows):
    k = pl.kernel(
        _pool_body,
        out_type=jax.ShapeDtypeStruct((NSC, G, D), jnp.float32),
        mesh=_mesh(),
        scratch_types=[pltpu.VMEM_SHARED((PG_ROWS, D), jnp.float32),
                       pltpu.VMEM((EMB_CH,), jnp.int32),
                       pltpu.VMEM((EMB_CH, D), jnp.float32),
                       pltpu.SemaphoreType.DMA],
        compiler_params=_sc_params(),
    )
    return k(h, batch_pad, zeros_rows)


def _conv_body(a_ref, h_ref, wr_ref, wo_ref, b_ref, o_ref):
    o_ref[...] = jnp.maximum(
        jnp.dot(a_ref[...], wr_ref[...],
                preferred_element_type=jnp.float32)
        + jnp.dot(h_ref[...], wo_ref[...],
                  preferred_element_type=jnp.float32)
        + b_ref[...], 0.0)


def _conv_tc(aggr, h, wr, wo, b):
    blk = 6272
    return pl.pallas_call(
        _conv_body,
        grid=(NP // blk,),
        in_specs=[pl.BlockSpec((blk, D), lambda i: (i, 0)),
                  pl.BlockSpec((blk, D), lambda i: (i, 0)),
                  pl.BlockSpec((D, D), lambda i: (0, 0)),
                  pl.BlockSpec((D, D), lambda i: (0, 0)),
                  pl.BlockSpec((1, D), lambda i: (0, 0))],
        out_specs=pl.BlockSpec((blk, D), lambda i: (i, 0)),
        out_shape=jax.ShapeDtypeStruct((NP, D), jnp.float32),
    )(aggr, h, wr, wo, b.reshape(1, D))


def _mlp_body(pp_ref, w0_ref, b0_ref, wl_ref, bl_ref, wout_ref, bout_ref,
              o_ref):
    p = pp_ref[0] + pp_ref[1]
    hh = jnp.maximum(
        jnp.dot(p, w0_ref[...], preferred_element_type=jnp.float32) + b0_ref[...], 0.0)
    for i in range(2):
        hh = jnp.maximum(
            jnp.dot(hh, wl_ref[i], preferred_element_type=jnp.float32) + bl_ref[i], 0.0)
    o_ref[...] = (jnp.dot(hh, wout_ref[...],
                          preferred_element_type=jnp.float32)
                  + bout_ref[...])


def _mlp_tc(pp, w0, b0, wl, bl, wout, bout):
    return pl.pallas_call(
        _mlp_body,
        out_shape=jax.ShapeDtypeStruct((G, 1), jnp.float32),
    )(pp, w0, b0.reshape(1, H), wl, bl.reshape(2, 1, H), wout,
      bout.reshape(1, 1))


def kernel(x, edge_index, batch, Wemb, Wrel, Wroot, bconv, W0, b0, Wl, bl,
           Wout, bout):
    x_pad = jnp.concatenate(
        [x.astype(jnp.int32), jnp.zeros((NP - N,), jnp.int32)])
    batch_pad = jnp.concatenate(
        [batch.astype(jnp.int32), jnp.full((NP - N,), G, jnp.int32)])
    src = edge_index[0].astype(jnp.int32)
    dst = edge_index[1].astype(jnp.int32)

    srcc, dstc, counts = _setup(src, dst)
    zeros_rows = jnp.zeros((TPT, D), jnp.float32)

    h = _embed(x_pad, Wemb)
    for i in range(L):
        aggr = _aggregate(h, srcc, dstc, counts, zeros_rows)
        h = _conv_tc(aggr, h, Wrel[i], Wroot[i], bconv[i])

    pp = _pool(h, batch_pad, zeros_rows)
    return _mlp_tc(pp, W0, b0, Wl, bl, Wout, bout)


# speedup vs baseline: 2.9715x; 1.0071x over previous
import dataclasses

import jax
import jax.numpy as jnp
from jax import lax
from jax.experimental import pallas as pl
from jax.experimental.pallas import tpu as pltpu
from jax.experimental.pallas import tpu_sc as plsc

N = 50000
E = 800000
G = 1024
D = 64
H = 128
L = 6

NSC = 2
NSUB = 16
NW = NSC * NSUB

NP = 50176
TPT = NP // NW
QSIZE = NP // 4
QPT = QSIZE // NSUB
TRASH = QSIZE
ACC_ROWS = QSIZE + 16
EPT = E // NSUB
SET_CH = 2000
PADM = 1024
FBUF = PADM + 32
W_Q = 50176
NBLKQ = W_Q // 128
CH = 128
EMB_CH = 112
PG_ROWS = G + 16


def _mesh():
    return plsc.VectorSubcoreMesh(
        core_axis_name="c", subcore_axis_name="s",
        num_cores=NSC, num_subcores=NSUB)


def _sc_params():
    return dataclasses.replace(
        pltpu.CompilerParams(), needs_layout_passes=False,
        use_tc_tiling_on_sc=False)


def _setup_body(src_hbm, dst_hbm, srcc_hbm, dstc_hbm, cnt_hbm,
                sbuf, dbuf, so0, do0, so1, do1, cbuf):
    c = lax.axis_index("c")
    s = lax.axis_index("s")
    r = c * NSUB + s
    base0 = (2 * c) * QSIZE
    base1 = (2 * c + 1) * QSIZE
    ebase = s * EPT

    def chunk_body(k, carry):
        pltpu.sync_copy(src_hbm.at[pl.ds(ebase + k * SET_CH, SET_CH)], sbuf)
        pltpu.sync_copy(dst_hbm.at[pl.ds(ebase + k * SET_CH, SET_CH)], dbuf)

        def vec_body(i, carry):
            ptr0, ptr1, cur0, cur1 = carry
            sv = sbuf[pl.ds(i * 16, 16)]
            dv = dbuf[pl.ds(i * 16, 16)]

            ld0 = dv - base0
            m0 = (ld0 >= 0) & (ld0 < QSIZE)
            plsc.store_compressed(so0.at[pl.ds(ptr0, 16)], sv, mask=m0)
            plsc.store_compressed(do0.at[pl.ds(ptr0, 16)], ld0, mask=m0)
            ptr0 = ptr0 + jnp.sum(m0.astype(jnp.int32))

            ld1 = dv - base1
            m1 = (ld1 >= 0) & (ld1 < QSIZE)
            plsc.store_compressed(so1.at[pl.ds(ptr1, 16)], sv, mask=m1)
            plsc.store_compressed(do1.at[pl.ds(ptr1, 16)], ld1, mask=m1)
            ptr1 = ptr1 + jnp.sum(m1.astype(jnp.int32))

            f0 = ptr0 >= PADM

            @pl.when(f0)
            def _():
                c0 = pl.multiple_of(cur0, PADM)
                pltpu.sync_copy(so0.at[pl.ds(0, PADM)],
                                srcc_hbm.at[r, 0, pl.ds(c0, PADM)])
                pltpu.sync_copy(do0.at[pl.ds(0, PADM)],
                                dstc_hbm.at[r, 0, pl.ds(c0, PADM)])
                so0[pl.ds(0, 16)] = so0[pl.ds(PADM, 16)]
                do0[pl.ds(0, 16)] = do0[pl.ds(PADM, 16)]

            ptr0 = jnp.where(f0, ptr0 - PADM, ptr0)
            cur0 = jnp.where(f0, cur0 + PADM, cur0)

            f1 = ptr1 >= PADM

            @pl.when(f1)
            def _():
                c1 = pl.multiple_of(cur1, PADM)
                pltpu.sync_copy(so1.at[pl.ds(0, PADM)],
                                srcc_hbm.at[r, 1, pl.ds(c1, PADM)])
                pltpu.sync_copy(do1.at[pl.ds(0, PADM)],
                                dstc_hbm.at[r, 1, pl.ds(c1, PADM)])
                so1[pl.ds(0, 16)] = so1[pl.ds(PADM, 16)]
                do1[pl.ds(0, 16)] = do1[pl.ds(PADM, 16)]

            ptr1 = jnp.where(f1, ptr1 - PADM, ptr1)
            cur1 = jnp.where(f1, cur1 + PADM, cur1)

            return ptr0, ptr1, cur0, cur1

        return lax.fori_loop(0, SET_CH // 16, vec_body, carry)

    z = jnp.int32(0)
    ptr0, ptr1, cur0, cur1 = lax.fori_loop(
        0, EPT // SET_CH, chunk_body, (z, z, z, z))

    lanes = lax.iota(jnp.int32, 16)
    zeros16 = jnp.zeros((16,), jnp.int32)
    trash16 = jnp.full((16,), TRASH, jnp.int32)

    def tail_body(k, _):
        pos = lanes + k * 16
        sc0 = so0[pl.ds(k * 16, 16)]
        dc0 = do0[pl.ds(k * 16, 16)]
        so0[pl.ds(k * 16, 16)] = jnp.where(pos >= ptr0, zeros16, sc0)
        do0[pl.ds(k * 16, 16)] = jnp.where(pos >= ptr0, trash16, dc0)
        sc1 = so1[pl.ds(k * 16, 16)]
        dc1 = do1[pl.ds(k * 16, 16)]
        so1[pl.ds(k * 16, 16)] = jnp.where(pos >= ptr1, zeros16, sc1)
        do1[pl.ds(k * 16, 16)] = jnp.where(pos >= ptr1, trash16, dc1)
        return 0

    lax.fori_loop(0, PADM // 16, tail_body, 0)

    cur0 = pl.multiple_of(cur0, PADM)
    cur1 = pl.multiple_of(cur1, PADM)
    pltpu.sync_copy(so0.at[pl.ds(0, PADM)],
                    srcc_hbm.at[r, 0, pl.ds(cur0, PADM)])
    pltpu.sync_copy(do0.at[pl.ds(0, PADM)],
                    dstc_hbm.at[r, 0, pl.ds(cur0, PADM)])
    pltpu.sync_copy(so1.at[pl.ds(0, PADM)],
                    srcc_hbm.at[r, 1, pl.ds(cur1, PADM)])
    pltpu.sync_copy(do1.at[pl.ds(0, PADM)],
                    dstc_hbm.at[r, 1, pl.ds(cur1, PADM)])

    cbuf[...] = jnp.full((16,), cur0 + PADM, jnp.int32)
    pltpu.sync_copy(cbuf, cnt_hbm.at[r, 0])
    cbuf[...] = jnp.full((16,), cur1 + PADM, jnp.int32)
    pltpu.sync_copy(cbuf, cnt_hbm.at[r, 1])


def _setup(src, dst):
    k = pl.kernel(
        _setup_body,
        out_type=(jax.ShapeDtypeStruct((NW, 2, W_Q), jnp.int32),
                  jax.ShapeDtypeStruct((NW, 2, W_Q), jnp.int32),
                  jax.ShapeDtypeStruct((NW, 2, 16), jnp.int32)),
        mesh=_mesh(),
        scratch_types=[pltpu.VMEM((SET_CH,), jnp.int32),
                       pltpu.VMEM((SET_CH,), jnp.int32),
                       pltpu.VMEM((FBUF,), jnp.int32),
                       pltpu.VMEM((FBUF,), jnp.int32),
                       pltpu.VMEM((FBUF,), jnp.int32),
                       pltpu.VMEM((FBUF,), jnp.int32),
                       pltpu.VMEM((16,), jnp.int32)],
        compiler_params=_sc_params(),
    )
    return k(src, dst)


def _embed_body(x_hbm, wemb_hbm, h_hbm, xi, rows, sem):
    c = lax.axis_index("c")
    s = lax.axis_index("s")
    base = (c * NSUB + s) * TPT

    @pl.loop(0, TPT // EMB_CH)
    def _(g):
        off = base + g * EMB_CH
        pltpu.sync_copy(x_hbm.at[pl.ds(off, EMB_CH)], xi)
        pltpu.async_copy(wemb_hbm.at[xi], rows, sem).wait()
        pltpu.sync_copy(rows, h_hbm.at[pl.ds(off, EMB_CH)])


def _embed(x_pad, wemb):
    k = pl.kernel(
        _embed_body,
        out_type=jax.ShapeDtypeStruct((NP, D), jnp.float32),
        mesh=_mesh(),
        scratch_types=[pltpu.VMEM((EMB_CH,), jnp.int32),
                       pltpu.VMEM((EMB_CH, D), jnp.float32),
                       pltpu.SemaphoreType.DMA],
        compiler_params=_sc_params(),
    )
    return k(x_pad, wemb)


def _aggr_body(h_hbm, srcc_hbm, dstc_hbm, cnt_hbm, zeros_hbm, out_hbm,
               acc, sidx, didx, rows, cnt, gsem0, gsem1, asem):
    c = lax.axis_index("c")
    s = lax.axis_index("s")
    r = c * NSUB + s

    for q in range(2):
        pltpu.sync_copy(zeros_hbm.at[pl.ds(0, QPT)],
                        acc.at[pl.ds(s * QPT, QPT)])
        plsc.subcore_barrier()

        pltpu.sync_copy(cnt_hbm.at[r, q], cnt)
        npairs = jnp.max(cnt[...]) // PADM

        def pair_body(p, carry):
            blk = p * 8
            pltpu.sync_copy(srcc_hbm.at[r, q, pl.ds(blk, 8)], sidx)
            pltpu.sync_copy(dstc_hbm.at[r, q, pl.ds(blk, 8)], didx)
            for j in range(8):
                pltpu.async_copy(h_hbm.at[sidx.at[j]], rows.at[j], gsem0)
            for j in range(8):
                pltpu.make_async_copy(h_hbm.at[sidx.at[j]], rows.at[j],
                                      gsem0).wait()
                pltpu.async_copy(rows.at[j], acc.at[didx.at[j]], asem,
                                 add=True)
            for j in range(8):
                pltpu.make_async_copy(rows.at[j], acc.at[didx.at[j]],
                                      asem).wait()
            return carry

        lax.fori_loop(0, npairs, pair_body, 0)
        plsc.subcore_barrier()
        pltpu.sync_copy(
            acc.at[pl.ds(s * QPT, QPT)],
            out_hbm.at[pl.ds((2 * c + q) * QSIZE + s * QPT, QPT)])


def _aggregate(h, srcc, dstc, counts, zeros_rows):
    k = pl.kernel(
        _aggr_body,
        out_type=jax.ShapeDtypeStruct((NP, D), jnp.float32),
        mesh=_mesh(),
        scratch_types=[pltpu.VMEM_SHARED((ACC_ROWS, D), jnp.float32),
                       pltpu.VMEM((8, CH), jnp.int32),
                       pltpu.VMEM((8, CH), jnp.int32),
                       pltpu.VMEM((8, CH, D), jnp.float32),
                       pltpu.VMEM((16,), jnp.int32),
                       pltpu.SemaphoreType.DMA,
                       pltpu.SemaphoreType.DMA,
                       pltpu.SemaphoreType.DMA],
        compiler_params=_sc_params(),
    )
    srcc4 = srcc.reshape(NW, 2, NBLKQ, CH)
    dstc4 = dstc.reshape(NW, 2, NBLKQ, CH)
    return k(h, srcc4, dstc4, counts, zeros_rows)


def _pool_body(h_hbm, batch_hbm, zeros_hbm, pp_hbm, pooled, bidx, rows, sem):
    c = lax.axis_index("c")
    s = lax.axis_index("s")
    base = c * (NP // 2) + s * TPT
    gpt = G // NSUB

    pltpu.sync_copy(zeros_hbm.at[pl.ds(0, gpt)],
                    pooled.at[pl.ds(s * gpt, gpt)])
    plsc.subcore_barrier()

    @pl.loop(0, TPT // EMB_CH)
    def _(g):
        off = base + g * EMB_CH
        pltpu.sync_copy(batch_hbm.at[pl.ds(off, EMB_CH)], bidx)
        pltpu.sync_copy(h_hbm.at[pl.ds(off, EMB_CH)], rows)
        pltpu.sync_copy(rows, pooled.at[bidx], add=True)

    plsc.subcore_barrier()
    pltpu.sync_copy(pooled.at[pl.ds(s * gpt, gpt)],
                    pp_hbm.at[c, pl.ds(s * gpt, gpt)])


def _pool(h, batch_pad, zeros_rows):
    k = pl.kernel(
        _pool_body,
        out_type=jax.ShapeDtypeStruct((NSC, G, D), jnp.float32),
        mesh=_mesh(),
        scratch_types=[pltpu.VMEM_SHARED((PG_ROWS, D), jnp.float32),
                       pltpu.VMEM((EMB_CH,), jnp.int32),
                       pltpu.VMEM((EMB_CH, D), jnp.float32),
                       pltpu.SemaphoreType.DMA],
        compiler_params=_sc_params(),
    )
    return k(h, batch_pad, zeros_rows)


def _conv_body(a_ref, h_ref, wr_ref, wo_ref, b_ref, o_ref):
    o_ref[...] = jnp.maximum(
        jnp.dot(a_ref[...], wr_ref[...],
                preferred_element_type=jnp.float32)
        + jnp.dot(h_ref[...], wo_ref[...],
                  preferred_element_type=jnp.float32)
        + b_ref[...], 0.0)


def _conv_tc(aggr, h, wr, wo, b):
    blk = 6272
    return pl.pallas_call(
        _conv_body,
        grid=(NP // blk,),
        in_specs=[pl.BlockSpec((blk, D), lambda i: (i, 0)),
                  pl.BlockSpec((blk, D), lambda i: (i, 0)),
                  pl.BlockSpec((D, D), lambda i: (0, 0)),
                  pl.BlockSpec((D, D), lambda i: (0, 0)),
                  pl.BlockSpec((1, D), lambda i: (0, 0))],
        out_specs=pl.BlockSpec((blk, D), lambda i: (i, 0)),
        out_shape=jax.ShapeDtypeStruct((NP, D), jnp.float32),
    )(aggr, h, wr, wo, b.reshape(1, D))


def _mlp_body(pp_ref, w0_ref, b0_ref, wl_ref, bl_ref, wout_ref, bout_ref,
              o_ref):
    p = pp_ref[0] + pp_ref[1]
    hh = jnp.maximum(
        jnp.dot(p, w0_ref[...], preferred_element_type=jnp.float32) + b0_ref[...], 0.0)
    for i in range(2):
        hh = jnp.maximum(
            jnp.dot(hh, wl_ref[i], preferred_element_type=jnp.float32) + bl_ref[i], 0.0)
    o_ref[...] = (jnp.dot(hh, wout_ref[...],
                          preferred_element_type=jnp.float32)
                  + bout_ref[...])


def _mlp_tc(pp, w0, b0, wl, bl, wout, bout):
    return pl.pallas_call(
        _mlp_body,
        out_shape=jax.ShapeDtypeStruct((G, 1), jnp.float32),
    )(pp, w0, b0.reshape(1, H), wl, bl.reshape(2, 1, H), wout,
      bout.reshape(1, 1))


def kernel(x, edge_index, batch, Wemb, Wrel, Wroot, bconv, W0, b0, Wl, bl,
           Wout, bout):
    x_pad = jnp.concatenate(
        [x.astype(jnp.int32), jnp.zeros((NP - N,), jnp.int32)])
    batch_pad = jnp.concatenate(
        [batch.astype(jnp.int32), jnp.full((NP - N,), G, jnp.int32)])
    src = edge_index[0].astype(jnp.int32)
    dst = edge_index[1].astype(jnp.int32)

    srcc, dstc, counts = _setup(src, dst)
    zeros_rows = jnp.zeros((TPT, D), jnp.float32)

    h = _embed(x_pad, Wemb)
    for i in range(L):
        aggr = _aggregate(h, srcc, dstc, counts, zeros_rows)
        h = _conv_tc(aggr, h, Wrel[i], Wroot[i], bconv[i])

    pp = _pool(h, batch_pad, zeros_rows)
    return _mlp_tc(pp, W0, b0, Wl, bl, Wout, bout)
